# baseline (speedup 1.0000x reference)
import functools

import numpy as np
import jax
import jax.numpy as jnp
from jax import lax
from jax.experimental import pallas as pl
from jax.experimental.pallas import tpu as pltpu
from jax.experimental.pallas import tpu_sc as plsc

VOCAB_SIZE = 1000000
D_DIM = 64
BATCH_N = 4096
SEQ_L = 200


def _pe_table() -> np.ndarray:
    pos = np.arange(SEQ_L)[:, np.newaxis].astype(np.float64)
    i = np.arange(D_DIM)[np.newaxis, :].astype(np.float64)
    angle_rates = 1.0 / np.power(10000, 2 * (i // 2) / np.float32(D_DIM))
    angle_rads = pos * angle_rates
    angle_rads[:, 0::2] = np.sin(angle_rads[:, 0::2])
    angle_rads[:, 1::2] = np.cos(angle_rads[:, 1::2])
    return np.asarray(angle_rads, dtype=np.float32)


_PE_CONST = _pe_table()

_INFO = plsc.get_sparse_core_info()
_NC, _NS = _INFO.num_cores, _INFO.num_subcores
NW = _NC * _NS

N_ROWS = BATCH_N * SEQ_L
PER_W = N_ROWS // NW
SEQ_PER_CHUNK = 4
CHUNK = SEQ_PER_CHUNK * SEQ_L
NCHUNK = PER_W // CHUNK
_SUBS = [(0, 128), (128, 128), (256, 128), (384, 128),
         (512, 128), (640, 128), (768, 32)]
LANES = 16
VECS_PER_ROW = D_DIM // LANES


def _sc_embed(table, idx_flat, pe):
    mesh = plsc.VectorSubcoreMesh(core_axis_name="c", subcore_axis_name="s")

    @functools.partial(
        pl.kernel,
        mesh=mesh,
        out_type=jax.ShapeDtypeStruct((N_ROWS, D_DIM), jnp.float32),
        scratch_types=[
            pltpu.VMEM((CHUNK,), jnp.int32),
            pltpu.VMEM((CHUNK, D_DIM), jnp.float32),
            pltpu.VMEM((SEQ_L, D_DIM), jnp.float32),
            pltpu.SemaphoreType.DMA,
        ],
        compiler_params=pltpu.CompilerParams(use_tc_tiling_on_sc=False),
    )
    def body(table_hbm, idx_hbm, pe_hbm, out_hbm, idx_v, rows_v, pe_v, sem):
        wid = lax.axis_index("s") * _NC + lax.axis_index("c")
        base = wid * PER_W
        pltpu.sync_copy(pe_hbm, pe_v)

        def chunk_body(g, carry):
            row0 = base + g * CHUNK
            pltpu.sync_copy(idx_hbm.at[pl.ds(row0, CHUNK)], idx_v)
            copies = []
            for off, ln in _SUBS:
                copies.append(
                    pltpu.async_copy(
                        table_hbm.at[idx_v.at[pl.ds(off, ln)]],
                        rows_v.at[pl.ds(off, ln)],
                        sem,
                    )
                )
            for cp in copies:
                cp.wait()

            def add_body(r, c2):
                for s in range(SEQ_PER_CHUNK):
                    row = s * SEQ_L + r
                    for j in range(VECS_PER_ROW):
                        sl = pl.ds(j * LANES, LANES)
                        rows_v[row, sl] = rows_v[row, sl] + pe_v[r, sl]
                return c2

            lax.fori_loop(0, SEQ_L, add_body, 0)
            pltpu.sync_copy(rows_v, out_hbm.at[pl.ds(row0, CHUNK)])
            return carry

        lax.fori_loop(0, NCHUNK, chunk_body, 0)

    return body(table, idx_flat, pe)


def kernel(inputs, table):
    idx_flat = inputs.reshape(-1)
    pe = jnp.asarray(_PE_CONST)
    out = _sc_embed(table, idx_flat, pe)
    return out.reshape(BATCH_N, SEQ_L, D_DIM)
